# baseline (speedup 1.0000x reference)
import functools
import math

import jax
import jax.numpy as jnp
from jax import lax
from jax.experimental import pallas as pl
from jax.experimental.pallas import tpu as pltpu
from jax.experimental.pallas import tpu_sc as plsc

_S = 64.0
_M2 = 0.5
_COS_M = math.cos(_M2)
_SIN_M = math.sin(_M2)
_THETA = math.cos(math.pi - _M2)
_SINMM = math.sin(math.pi - _M2) * _M2

_B = 1024
_V = 100000

_NC = 2
_NS = 16
_L = 16
_NW = _NC * _NS
_RPW = _B // _NW

_CW = 6272
_VAL = (_V // 128) * 128
_VT = _V - _VAL
_CHUNKS = [(c0, min(_CW, _VAL - c0)) for c0 in range(0, _VAL, _CW)]

_GATHER_DNUMS = lax.GatherDimensionNumbers(
    offset_dims=(), collapsed_slice_dims=(0,), start_index_map=(0,))


def _sqrt16(a):
    s = 0.5 + 0.5 * a
    for _ in range(7):
        s = 0.5 * (s + a / s)
    return s


def _sc_body(logits_hbm, labels_hbm, out_hbm,
             lab_v, chunk_v, fixp_v, bufs, tail_b,
             sem_fr, sem_fw, sem_l0, sem_l1, sem_s0, sem_s1):
    wid = lax.axis_index("s") * _NC + lax.axis_index("c")
    base = wid * _RPW
    pltpu.sync_copy(labels_hbm.at[pl.ds(base, _RPW)], lab_v)

    fix_reads = []
    for j in range(_RPW // _L):
        labc = jnp.maximum(lab_v[pl.ds(j * _L, _L)], 0)
        for i in range(_L):
            r = j * _L + i
            c0 = pl.multiple_of((labc[i] // _L) * _L, _L)
            fix_reads.append(
                pltpu.async_copy(logits_hbm.at[base + r, pl.ds(c0, _L)],
                                 chunk_v.at[pl.ds(r * _L, _L)], sem_fr))

    sem_l = (sem_l0, sem_l1)
    sem_s = (sem_s0, sem_s1)
    tasks = [(tr, c0, cw) for tr in range(_RPW // 8) for (c0, cw) in _CHUNKS]
    n = len(tasks)

    def mk_load(k):
        tr, c0, cw = tasks[k]
        b = k % 2
        return pltpu.make_async_copy(
            logits_hbm.at[pl.ds(base + tr * 8, 8), pl.ds(c0, cw)],
            bufs.at[b, :, pl.ds(0, cw)], sem_l[b])

    def mk_store(k):
        tr, c0, cw = tasks[k]
        b = k % 2
        return pltpu.make_async_copy(
            bufs.at[b, :, pl.ds(0, cw)],
            out_hbm.at[pl.ds(base + tr * 8, 8), pl.ds(c0, cw)], sem_s[b])

    mk_load(0).start()
    for k in range(n):
        b = k % 2
        if k + 1 < n:
            if k >= 1:
                mk_store(k - 1).wait()
            mk_load(k + 1).start()
        mk_load(k).wait()
        cw = tasks[k][2]

        def body(i, carry):
            off = pl.multiple_of(i * _L, _L)
            for row in range(8):
                bufs[b, row, pl.ds(off, _L)] = bufs[b, row, pl.ds(off, _L)] * _S
            return carry

        lax.fori_loop(0, cw // _L, body, 0)
        mk_store(k).start()
    if n >= 2:
        mk_store(n - 2).wait()
    mk_store(n - 1).wait()

    for tr in range(_RPW // 8):
        pltpu.sync_copy(
            logits_hbm.at[pl.ds(base + tr * 8, 8), pl.ds(_VAL, _VT)], tail_b)
        for row in range(8):
            for q in range(_VT // _L):
                tail_b[row, pl.ds(q * _L, _L)] = \
                    tail_b[row, pl.ds(q * _L, _L)] * _S
        pltpu.sync_copy(
            tail_b, out_hbm.at[pl.ds(base + tr * 8, 8), pl.ds(_VAL, _VT)])

    for c in fix_reads:
        c.wait()
    lanes16 = lax.iota(jnp.int32, _L)
    fix_writes = []
    for j in range(_RPW // _L):
        labv = lab_v[pl.ds(j * _L, _L)]
        labc = jnp.maximum(labv, 0)
        lanev = labc % _L
        tacc = jnp.zeros((_L,), jnp.float32)
        for i in range(_L):
            r = j * _L + i
            chunk = chunk_v[pl.ds(r * _L, _L)]
            vals = lax.gather(chunk, lanev[:, None], _GATHER_DNUMS,
                              slice_sizes=(1,),
                              mode=lax.GatherScatterMode.PROMISE_IN_BOUNDS)
            tacc = tacc + jnp.where(lanes16 == i, vals, 0.0)
        t = tacc
        sin_t = _sqrt16(1.0 - t * t)
        cos_theta_m = t * _COS_M - sin_t * _SIN_M
        f = jnp.where(t > _THETA, cos_theta_m, t - _SINMM)
        upd = jnp.where(labv >= 0, f, t) * _S
        for i in range(_L):
            r = j * _L + i
            scaled = chunk_v[pl.ds(r * _L, _L)] * _S
            iconst = jnp.full((_L,), i, jnp.int32)
            lane_b = lax.gather(lanev, iconst[:, None], _GATHER_DNUMS,
                                slice_sizes=(1,),
                                mode=lax.GatherScatterMode.PROMISE_IN_BOUNDS)
            upd_b = lax.gather(upd, iconst[:, None], _GATHER_DNUMS,
                               slice_sizes=(1,),
                               mode=lax.GatherScatterMode.PROMISE_IN_BOUNDS)
            newc = jnp.where(lanes16 == lane_b, upd_b, scaled)
            fixp_v[pl.ds(r * _L, _L)] = newc
            c0 = pl.multiple_of((labc[i] // _L) * _L, _L)
            fix_writes.append(
                pltpu.async_copy(fixp_v.at[pl.ds(r * _L, _L)],
                                 out_hbm.at[base + r, pl.ds(c0, _L)], sem_fw))
    for c in fix_writes:
        c.wait()


@functools.cache
def _sc_kernel():
    return functools.partial(
        pl.kernel,
        mesh=plsc.VectorSubcoreMesh(core_axis_name="c", subcore_axis_name="s"),
        out_type=jax.ShapeDtypeStruct((_B, _V), jnp.float32),
        scratch_types=[
            pltpu.VMEM((_RPW,), jnp.int32),
            pltpu.VMEM((_RPW * _L,), jnp.float32),
            pltpu.VMEM((_RPW * _L,), jnp.float32),
            pltpu.VMEM((2, 8, _CW), jnp.float32),
            pltpu.VMEM((8, _VT), jnp.float32),
            pltpu.SemaphoreType.DMA,
            pltpu.SemaphoreType.DMA,
            pltpu.SemaphoreType.DMA,
            pltpu.SemaphoreType.DMA,
            pltpu.SemaphoreType.DMA,
            pltpu.SemaphoreType.DMA,
        ],
    )(_sc_body)


def kernel(logits, labels):
    return _sc_kernel()(logits, labels)


# speedup vs baseline: 1.0934x; 1.0934x over previous
import functools
import math

import jax
import jax.numpy as jnp
from jax import lax
from jax.experimental import pallas as pl
from jax.experimental.pallas import tpu as pltpu
from jax.experimental.pallas import tpu_sc as plsc

_S = 64.0
_M2 = 0.5
_COS_M = math.cos(_M2)
_SIN_M = math.sin(_M2)
_THETA = math.cos(math.pi - _M2)
_SINMM = math.sin(math.pi - _M2) * _M2

_B = 1024
_V = 100000

_NC = 2
_NS = 16
_L = 16
_NW = _NC * _NS
_RPW = _B // _NW

_BR = 16


def _sc_gather_body(logits_hbm, labels_hbm, out_hbm,
                    lab_v, chunk_v, t_v, sem):
    wid = lax.axis_index("s") * _NC + lax.axis_index("c")
    base = wid * _RPW
    pltpu.sync_copy(labels_hbm.at[pl.ds(base, _RPW)], lab_v)
    copies = []
    for j in range(_RPW // _L):
        labv = jnp.maximum(lab_v[pl.ds(j * _L, _L)], 0)
        for i in range(_L):
            lab = labv[i]
            c0 = pl.multiple_of((lab // _L) * _L, _L)
            r = j * _L + i
            copies.append(
                pltpu.async_copy(logits_hbm.at[base + r, pl.ds(c0, _L)],
                                 chunk_v.at[pl.ds(r * _L, _L)], sem))
    for c in copies:
        c.wait()
    lanes16 = lax.iota(jnp.int32, _L)
    for j in range(_RPW // _L):
        lanev = jnp.maximum(lab_v[pl.ds(j * _L, _L)], 0) % _L
        tacc = jnp.zeros((_L,), jnp.float32)
        for i in range(_L):
            r = j * _L + i
            chunk = chunk_v[pl.ds(r * _L, _L)]
            vals = lax.gather(
                chunk, lanev[:, None],
                lax.GatherDimensionNumbers(offset_dims=(),
                                           collapsed_slice_dims=(0,),
                                           start_index_map=(0,)),
                slice_sizes=(1,),
                mode=lax.GatherScatterMode.PROMISE_IN_BOUNDS)
            tacc = tacc + jnp.where(lanes16 == i, vals, 0.0)
        t_v[pl.ds(j * _L, _L)] = tacc
    pltpu.sync_copy(t_v, out_hbm.at[pl.ds(base, _RPW)])


@functools.cache
def _sc_gather():
    return functools.partial(
        pl.kernel,
        mesh=plsc.VectorSubcoreMesh(core_axis_name="c", subcore_axis_name="s"),
        out_type=jax.ShapeDtypeStruct((_B,), jnp.float32),
        scratch_types=[
            pltpu.VMEM((_RPW,), jnp.int32),
            pltpu.VMEM((_RPW * _L,), jnp.float32),
            pltpu.VMEM((_RPW,), jnp.float32),
            pltpu.SemaphoreType.DMA,
        ],
    )(_sc_gather_body)


def _merge_body(lab_ref, t_ref, x_ref, o_ref):
    x = x_ref[...]
    lab = lab_ref[...]
    t = t_ref[...]
    sin_t = jnp.sqrt(1.0 - t * t)
    cos_theta_m = t * _COS_M - sin_t * _SIN_M
    f = jnp.where(t > _THETA, cos_theta_m, t - _SINMM)
    upd = jnp.where(lab >= 0, f, t)
    cols = lax.broadcasted_iota(jnp.int32, x.shape, 1)
    o_ref[...] = _S * jnp.where(cols == lab, upd, x)


def kernel(logits, labels):
    b, v = logits.shape
    t = _sc_gather()(logits, labels)
    return pl.pallas_call(
        _merge_body,
        grid=(b // _BR,),
        in_specs=[
            pl.BlockSpec((_BR, 1), lambda i: (i, 0)),
            pl.BlockSpec((_BR, 1), lambda i: (i, 0)),
            pl.BlockSpec((_BR, v), lambda i: (i, 0)),
        ],
        out_specs=pl.BlockSpec((_BR, v), lambda i: (i, 0)),
        out_shape=jax.ShapeDtypeStruct((b, v), jnp.float32),
    )(labels.reshape(b, 1), t.reshape(b, 1), logits)
